# baseline (speedup 1.0000x reference)
import dataclasses
import functools

import jax
import jax.numpy as jnp
from jax import lax
from jax.experimental import pallas as pl
from jax.experimental.pallas import tpu as pltpu
from jax.experimental.pallas import tpu_sc as plsc

N = 10000
E = 320000
F = 128
H = 128
C = 40

NC = 2
NS = 16
G = 128
KB = 40
K = KB * (-(-E // (NC * NS * G * KB)))
E_PAD = NC * NS * K * G
TRASH = N
N_ACC = 10240
RPS = N_ACC // NS
W2P = 64

_mesh = plsc.VectorSubcoreMesh(core_axis_name="c", subcore_axis_name="s")

_cp_no_layout = pltpu.CompilerParams()
if "needs_layout_passes" in pltpu.CompilerParams.__dataclass_fields__:
    _cp_no_layout = dataclasses.replace(_cp_no_layout, needs_layout_passes=False)


def _seg_cnt(dst4):
    @functools.partial(
        pl.kernel,
        out_type=jax.ShapeDtypeStruct((NC, N_ACC), jnp.float32),
        mesh=_mesh,
        compiler_params=_cp_no_layout,
        scratch_types=[
            pltpu.VMEM((K, G), jnp.int32),
            pltpu.VMEM((1, N_ACC), jnp.float32),
            pltpu.VMEM((NS, RPS), jnp.float32),
            pltpu.VMEM((1, RPS), jnp.float32),
            pltpu.VMEM_SHARED((NS, N_ACC), jnp.float32),
        ],
    )
    def k(dst_hbm, cnt_out, dst_v, hist_v, red_v, out_v, stage_sh):
        c = lax.axis_index("c")
        s = lax.axis_index("s")
        r0 = s * RPS

        pltpu.sync_copy(dst_hbm.at[c, s], dst_v)

        @pl.loop(0, N_ACC // 16)
        def _(i):
            hist_v[0, pl.ds(i * 16, 16)] = jnp.zeros((16,), jnp.float32)

        ones = jnp.full((16,), 1.0, jnp.float32)
        zrow = jnp.zeros((16,), jnp.int32)

        @pl.loop(0, K)
        def _(j):
            row = dst_v.at[j]
            for k2 in range(G // 16):
                idx = row[pl.ds(k2 * 16, 16)]
                plsc.addupdate_scatter(hist_v, [zrow, idx], ones)

        pltpu.sync_copy(hist_v, stage_sh.at[pl.ds(s, 1)])
        plsc.subcore_barrier()

        for r in range(NS):
            pltpu.sync_copy(stage_sh.at[pl.ds(r, 1), pl.ds(r0, RPS)],
                            red_v.at[pl.ds(r, 1)])

        @pl.loop(0, RPS // 16)
        def _(i):
            sl = pl.ds(i * 16, 16)
            acc = red_v[0, sl]
            for r in range(1, NS):
                acc = acc + red_v[r, sl]
            out_v[0, sl] = acc

        pltpu.sync_copy(out_v, cnt_out.at[pl.ds(c, 1), pl.ds(r0, RPS)])

    return k(dst4)


def _seg_sum(data, src4, dst4, zacc, width):
    @functools.partial(
        pl.kernel,
        out_type=jax.ShapeDtypeStruct((NC, N_ACC, width), jnp.float32),
        mesh=_mesh,
        scratch_types=[
            pltpu.VMEM((K, G), jnp.int32),
            pltpu.VMEM((K, G), jnp.int32),
            pltpu.VMEM((G, width), jnp.float32),
            pltpu.VMEM_SHARED((N_ACC, width), jnp.float32),
            pltpu.SemaphoreType.DMA,
        ],
    )
    def k(d_hbm, src_hbm, dst_hbm, zacc_hbm, acc_out,
          src_v, dst_v, rows_v, acc_sh, sem):
        c = lax.axis_index("c")
        s = lax.axis_index("s")
        r0 = s * RPS

        pltpu.sync_copy(zacc_hbm.at[pl.ds(r0, RPS)], acc_sh.at[pl.ds(r0, RPS)])
        pltpu.sync_copy(src_hbm.at[c, s], src_v)
        pltpu.sync_copy(dst_hbm.at[c, s], dst_v)

        plsc.subcore_barrier()

        @pl.loop(0, K)
        def _(j):
            pltpu.async_copy(d_hbm.at[src_v.at[j]], rows_v, sem).wait()
            pltpu.sync_copy(rows_v, acc_sh.at[dst_v.at[j]], add=True)

        plsc.subcore_barrier()

        pltpu.sync_copy(acc_sh.at[pl.ds(r0, RPS)],
                        acc_out.at[c, pl.ds(r0, RPS)])

    return k(data, src4, dst4, zacc)


_BN = 1000


def _tc1_body(x_ref, a1_ref, cnt_ref, w1s_ref, w1n_ref, b1_ref,
              w2c_ref, b2c_ref, u_ref):
    cnt = jnp.sum(cnt_ref[...], axis=1, keepdims=True)
    mean1 = (a1_ref[0] + a1_ref[1]) / jnp.maximum(cnt, 1.0)
    s1 = jnp.dot(x_ref[...], w1s_ref[...], preferred_element_type=jnp.float32)
    m1 = jnp.dot(mean1, w1n_ref[...], preferred_element_type=jnp.float32)
    h = jax.nn.relu(s1 + m1 + b1_ref[...])
    u = jnp.dot(h, w2c_ref[...], preferred_element_type=jnp.float32)
    u_ref[...] = u + b2c_ref[...]


def _tc1(x, a1, cnt16, W1_self, W1_neigh, b1, W2cat, b2cat):
    grid = (N // _BN,)
    return pl.pallas_call(
        _tc1_body,
        grid=grid,
        in_specs=[
            pl.BlockSpec((_BN, F), lambda i: (i, 0)),
            pl.BlockSpec((NC, _BN, F), lambda i: (0, i, 0)),
            pl.BlockSpec((_BN, NC), lambda i: (i, 0)),
            pl.BlockSpec((F, H), lambda i: (0, 0)),
            pl.BlockSpec((F, H), lambda i: (0, 0)),
            pl.BlockSpec((1, H), lambda i: (0, 0)),
            pl.BlockSpec((H, 2 * W2P), lambda i: (0, 0)),
            pl.BlockSpec((1, 2 * W2P), lambda i: (0, 0)),
        ],
        out_specs=pl.BlockSpec((_BN, 2 * W2P), lambda i: (i, 0)),
        out_shape=jax.ShapeDtypeStruct((N, 2 * W2P), jnp.float32),
    )(x, a1, cnt16, W1_self, W1_neigh, b1, W2cat, b2cat)


def _tc2_body(u_ref, a2_ref, cnt_ref, out_ref):
    cnt = jnp.sum(cnt_ref[...], axis=1, keepdims=True)
    mean2 = (a2_ref[0, :, :W2P] + a2_ref[1, :, :W2P]) / jnp.maximum(cnt, 1.0)
    t = u_ref[:, W2P:] + mean2
    col = lax.broadcasted_iota(jnp.int32, (_BN, W2P), 1)
    valid = col < C
    tm = jnp.where(valid, t, -jnp.inf)
    m = jnp.max(tm, axis=-1, keepdims=True)
    e = jnp.where(valid, jnp.exp(t - m), 0.0)
    lse = jnp.log(jnp.sum(e, axis=-1, keepdims=True))
    out_ref[...] = (t - m - lse)[:, :C]


def _tc2(u, a2, cnt16):
    grid = (N // _BN,)
    return pl.pallas_call(
        _tc2_body,
        grid=grid,
        in_specs=[
            pl.BlockSpec((_BN, 2 * W2P), lambda i: (i, 0)),
            pl.BlockSpec((NC, _BN, 2 * W2P), lambda i: (0, i, 0)),
            pl.BlockSpec((_BN, NC), lambda i: (i, 0)),
        ],
        out_specs=pl.BlockSpec((_BN, C), lambda i: (i, 0)),
        out_shape=jax.ShapeDtypeStruct((N, C), jnp.float32),
    )(u, a2, cnt16)


def kernel(x, adj, W1_self, W1_neigh, b1, W2_self, W2_neigh, b2):
    src, dst = adj[0], adj[1]
    pad = E_PAD - E
    src_p = jnp.concatenate([src, jnp.zeros((pad,), jnp.int32)])
    dst_p = jnp.concatenate([dst, jnp.full((pad,), TRASH, jnp.int32)])
    src4 = src_p.reshape(NC, NS, K, G)
    dst4 = dst_p.reshape(NC, NS, K, G)

    zacc1 = jnp.zeros((N_ACC, F), jnp.float32)

    cnt2 = _seg_cnt(dst4).T[:N]
    a1 = _seg_sum(x, src4, dst4, zacc1, F)[:, :N]

    W2cat = jnp.concatenate([
        jnp.pad(W2_neigh, ((0, 0), (0, W2P - C))),
        jnp.pad(W2_self, ((0, 0), (0, W2P - C))),
    ], axis=1)
    b2cat = jnp.concatenate(
        [jnp.zeros((W2P,), jnp.float32), jnp.pad(b2, (0, W2P - C))]
    ).reshape(1, 2 * W2P)

    u = _tc1(x, a1, cnt2, W1_self, W1_neigh, b1.reshape(1, H),
             W2cat, b2cat)

    a2 = _seg_sum(u, src4, dst4, zacc1, 2 * W2P)[:, :N]

    return _tc2(u, a2, cnt2)


# speedup vs baseline: 1.4863x; 1.4863x over previous
import dataclasses
import functools

import jax
import jax.numpy as jnp
from jax import lax
from jax.experimental import pallas as pl
from jax.experimental.pallas import tpu as pltpu
from jax.experimental.pallas import tpu_sc as plsc

N = 10000
E = 320000
F = 128
H = 128
C = 40

NC = 2
NS = 16
G = 128
K = -(-E // (NC * NS * G))
E_PAD = NC * NS * K * G
TRASH = N
N_ACC = 10240
RPS = N_ACC // NS
W2P = 64

_mesh = plsc.VectorSubcoreMesh(core_axis_name="c", subcore_axis_name="s")

_cp_no_layout = pltpu.CompilerParams()
if "needs_layout_passes" in pltpu.CompilerParams.__dataclass_fields__:
    _cp_no_layout = dataclasses.replace(_cp_no_layout, needs_layout_passes=False)


def _seg_cnt(dst4):
    @functools.partial(
        pl.kernel,
        out_type=jax.ShapeDtypeStruct((NC, N_ACC), jnp.float32),
        mesh=_mesh,
        compiler_params=_cp_no_layout,
        scratch_types=[
            pltpu.VMEM((K, G), jnp.int32),
            pltpu.VMEM((1, N_ACC), jnp.float32),
            pltpu.VMEM((NS, RPS), jnp.float32),
            pltpu.VMEM((1, RPS), jnp.float32),
            pltpu.VMEM_SHARED((NS, N_ACC), jnp.float32),
        ],
    )
    def k(dst_hbm, cnt_out, dst_v, hist_v, red_v, out_v, stage_sh):
        c = lax.axis_index("c")
        s = lax.axis_index("s")
        r0 = s * RPS

        pltpu.sync_copy(dst_hbm.at[c, s], dst_v)

        @pl.loop(0, N_ACC // 16)
        def _(i):
            hist_v[0, pl.ds(i * 16, 16)] = jnp.zeros((16,), jnp.float32)

        ones = jnp.full((16,), 1.0, jnp.float32)
        zrow = jnp.zeros((16,), jnp.int32)

        @pl.loop(0, K)
        def _(j):
            row = dst_v.at[j]
            for k2 in range(G // 16):
                idx = row[pl.ds(k2 * 16, 16)]
                plsc.addupdate_scatter(hist_v, [zrow, idx], ones)

        pltpu.sync_copy(hist_v, stage_sh.at[pl.ds(s, 1)])
        plsc.subcore_barrier()

        for r in range(NS):
            pltpu.sync_copy(stage_sh.at[pl.ds(r, 1), pl.ds(r0, RPS)],
                            red_v.at[pl.ds(r, 1)])

        @pl.loop(0, RPS // 16)
        def _(i):
            sl = pl.ds(i * 16, 16)
            acc = red_v[0, sl]
            for r in range(1, NS):
                acc = acc + red_v[r, sl]
            out_v[0, sl] = acc

        pltpu.sync_copy(out_v, cnt_out.at[pl.ds(c, 1), pl.ds(r0, RPS)])

    return k(dst4)


def _seg_sum(data, src4, dst4, zacc, width):
    @functools.partial(
        pl.kernel,
        out_type=jax.ShapeDtypeStruct((NC, N_ACC, width), jnp.float32),
        mesh=_mesh,
        scratch_types=[
            pltpu.VMEM((K, G), jnp.int32),
            pltpu.VMEM((K, G), jnp.int32),
            pltpu.VMEM((G, width), jnp.float32),
            pltpu.VMEM_SHARED((N_ACC, width), jnp.float32),
            pltpu.SemaphoreType.DMA,
        ],
    )
    def k(d_hbm, src_hbm, dst_hbm, zacc_hbm, acc_out,
          src_v, dst_v, rows_v, acc_sh, sem):
        c = lax.axis_index("c")
        s = lax.axis_index("s")
        r0 = s * RPS

        pltpu.sync_copy(zacc_hbm.at[pl.ds(r0, RPS)], acc_sh.at[pl.ds(r0, RPS)])
        pltpu.sync_copy(src_hbm.at[c, s], src_v)
        pltpu.sync_copy(dst_hbm.at[c, s], dst_v)

        plsc.subcore_barrier()

        @pl.loop(0, K)
        def _(j):
            pltpu.async_copy(d_hbm.at[src_v.at[j]], rows_v, sem).wait()
            pltpu.sync_copy(rows_v, acc_sh.at[dst_v.at[j]], add=True)

        plsc.subcore_barrier()

        pltpu.sync_copy(acc_sh.at[pl.ds(r0, RPS)],
                        acc_out.at[c, pl.ds(r0, RPS)])

    return k(data, src4, dst4, zacc)


_BN = 1000


def _tc1_body(x_ref, a1_ref, cnt_ref, w1s_ref, w1n_ref, b1_ref,
              w2c_ref, b2c_ref, u_ref):
    cnt = cnt_ref[:, 0:1] + cnt_ref[:, 1:2]
    mean1 = (a1_ref[0] + a1_ref[1]) / jnp.maximum(cnt, 1.0)
    s1 = jnp.dot(x_ref[...], w1s_ref[...], preferred_element_type=jnp.float32)
    m1 = jnp.dot(mean1, w1n_ref[...], preferred_element_type=jnp.float32)
    h = jax.nn.relu(s1 + m1 + b1_ref[...])
    u = jnp.dot(h, w2c_ref[...], preferred_element_type=jnp.float32)
    u_ref[...] = u + b2c_ref[...]


def _tc1(x, a1, cnt16, W1_self, W1_neigh, b1, W2cat, b2cat):
    grid = (N // _BN,)
    return pl.pallas_call(
        _tc1_body,
        grid=grid,
        in_specs=[
            pl.BlockSpec((_BN, F), lambda i: (i, 0)),
            pl.BlockSpec((NC, _BN, F), lambda i: (0, i, 0)),
            pl.BlockSpec((_BN, NC), lambda i: (i, 0)),
            pl.BlockSpec((F, H), lambda i: (0, 0)),
            pl.BlockSpec((F, H), lambda i: (0, 0)),
            pl.BlockSpec((1, H), lambda i: (0, 0)),
            pl.BlockSpec((H, 2 * W2P), lambda i: (0, 0)),
            pl.BlockSpec((1, 2 * W2P), lambda i: (0, 0)),
        ],
        out_specs=pl.BlockSpec((_BN, 2 * W2P), lambda i: (i, 0)),
        out_shape=jax.ShapeDtypeStruct((N, 2 * W2P), jnp.float32),
    )(x, a1, cnt16, W1_self, W1_neigh, b1, W2cat, b2cat)


def _tc2_body(u_ref, a2_ref, cnt_ref, out_ref):
    cnt = cnt_ref[:, 0:1] + cnt_ref[:, 1:2]
    mean2 = (a2_ref[0, :, :W2P] + a2_ref[1, :, :W2P]) / jnp.maximum(cnt, 1.0)
    t = u_ref[:, W2P:] + mean2
    col = lax.broadcasted_iota(jnp.int32, (_BN, W2P), 1)
    valid = col < C
    tm = jnp.where(valid, t, -jnp.inf)
    m = jnp.max(tm, axis=-1, keepdims=True)
    e = jnp.where(valid, jnp.exp(t - m), 0.0)
    lse = jnp.log(jnp.sum(e, axis=-1, keepdims=True))
    out_ref[...] = (t - m - lse)[:, :C]


def _tc2(u, a2, cnt16):
    grid = (N // _BN,)
    return pl.pallas_call(
        _tc2_body,
        grid=grid,
        in_specs=[
            pl.BlockSpec((_BN, 2 * W2P), lambda i: (i, 0)),
            pl.BlockSpec((NC, _BN, 2 * W2P), lambda i: (0, i, 0)),
            pl.BlockSpec((_BN, NC), lambda i: (i, 0)),
        ],
        out_specs=pl.BlockSpec((_BN, C), lambda i: (i, 0)),
        out_shape=jax.ShapeDtypeStruct((N, C), jnp.float32),
    )(u, a2, cnt16)


def kernel(x, adj, W1_self, W1_neigh, b1, W2_self, W2_neigh, b2):
    src, dst = adj[0], adj[1]
    pad = E_PAD - E
    src_p = jnp.concatenate([src, jnp.zeros((pad,), jnp.int32)])
    dst_p = jnp.concatenate([dst, jnp.full((pad,), TRASH, jnp.int32)])
    src4 = src_p.reshape(NC, NS, K, G)
    dst4 = dst_p.reshape(NC, NS, K, G)

    zacc1 = jnp.zeros((N_ACC, F), jnp.float32)

    cnt2 = _seg_cnt(dst4).T[:N]
    a1 = _seg_sum(x, src4, dst4, zacc1, F)[:, :N]

    W2cat = jnp.concatenate([
        jnp.pad(W2_neigh, ((0, 0), (0, W2P - C))),
        jnp.pad(W2_self, ((0, 0), (0, W2P - C))),
    ], axis=1)
    b2cat = jnp.concatenate(
        [jnp.zeros((W2P,), jnp.float32), jnp.pad(b2, (0, W2P - C))]
    ).reshape(1, 2 * W2P)

    u = _tc1(x, a1, cnt2, W1_self, W1_neigh, b1.reshape(1, H),
             W2cat, b2cat)

    a2 = _seg_sum(u, src4, dst4, zacc1, 2 * W2P)[:, :N]

    return _tc2(u, a2, cnt2)


# speedup vs baseline: 2.4932x; 1.6774x over previous
import dataclasses
import functools

import jax
import jax.numpy as jnp
from jax import lax
from jax.experimental import pallas as pl
from jax.experimental.pallas import tpu as pltpu
from jax.experimental.pallas import tpu_sc as plsc

N = 10000
E = 320000
F = 128
H = 128
C = 40

NC = 2
NS = 16
G = 128
K = -(-E // (NC * NS * G))
E_PAD = NC * NS * K * G
TRASH = N
N_ACC = 10240
RPS = N_ACC // NS
W2P = 64

_mesh = plsc.VectorSubcoreMesh(core_axis_name="c", subcore_axis_name="s")

_cp_no_layout = pltpu.CompilerParams()
if "needs_layout_passes" in pltpu.CompilerParams.__dataclass_fields__:
    _cp_no_layout = dataclasses.replace(_cp_no_layout, needs_layout_passes=False)


def _seg_cnt(dst4):
    @functools.partial(
        pl.kernel,
        out_type=jax.ShapeDtypeStruct((NC, N_ACC), jnp.float32),
        mesh=_mesh,
        compiler_params=_cp_no_layout,
        scratch_types=[
            pltpu.VMEM((K, G), jnp.int32),
            pltpu.VMEM((1, N_ACC), jnp.float32),
            pltpu.VMEM((NS, RPS), jnp.float32),
            pltpu.VMEM((1, RPS), jnp.float32),
            pltpu.VMEM_SHARED((NS, N_ACC), jnp.float32),
        ],
    )
    def k(dst_hbm, cnt_out, dst_v, hist_v, red_v, out_v, stage_sh):
        c = lax.axis_index("c")
        s = lax.axis_index("s")
        r0 = s * RPS

        pltpu.sync_copy(dst_hbm.at[c, s], dst_v)

        @pl.loop(0, N_ACC // 16)
        def _(i):
            hist_v[0, pl.ds(i * 16, 16)] = jnp.zeros((16,), jnp.float32)

        ones = jnp.full((16,), 1.0, jnp.float32)
        zrow = jnp.zeros((16,), jnp.int32)

        @pl.loop(0, K)
        def _(j):
            row = dst_v.at[j]
            for k2 in range(G // 16):
                idx = row[pl.ds(k2 * 16, 16)]
                plsc.addupdate_scatter(hist_v, [zrow, idx], ones)

        pltpu.sync_copy(hist_v, stage_sh.at[pl.ds(s, 1)])
        plsc.subcore_barrier()

        for r in range(NS):
            pltpu.sync_copy(stage_sh.at[pl.ds(r, 1), pl.ds(r0, RPS)],
                            red_v.at[pl.ds(r, 1)])

        @pl.loop(0, RPS // 16)
        def _(i):
            sl = pl.ds(i * 16, 16)
            acc = red_v[0, sl]
            for r in range(1, NS):
                acc = acc + red_v[r, sl]
            out_v[0, sl] = acc

        pltpu.sync_copy(out_v, cnt_out.at[pl.ds(c, 1), pl.ds(r0, RPS)])

    return k(dst4)


def _seg_sum(data, src4, dst4, zacc, width):
    @functools.partial(
        pl.kernel,
        out_type=jax.ShapeDtypeStruct((NC, N_ACC, width), jnp.float32),
        mesh=_mesh,
        scratch_types=[
            pltpu.VMEM((K, G), jnp.int32),
            pltpu.VMEM((K, G), jnp.int32),
            pltpu.VMEM((G, width), jnp.float32),
            pltpu.VMEM_SHARED((N_ACC, width), jnp.float32),
            pltpu.SemaphoreType.DMA,
        ],
    )
    def k(d_hbm, src_hbm, dst_hbm, zacc_hbm, acc_out,
          src_v, dst_v, rows_v, acc_sh, sem):
        c = lax.axis_index("c")
        s = lax.axis_index("s")
        r0 = s * RPS

        pltpu.sync_copy(zacc_hbm.at[pl.ds(r0, RPS)], acc_sh.at[pl.ds(r0, RPS)])
        pltpu.sync_copy(src_hbm.at[c, s], src_v)
        pltpu.sync_copy(dst_hbm.at[c, s], dst_v)

        plsc.subcore_barrier()

        @pl.loop(0, K)
        def _(j):
            pltpu.async_copy(d_hbm.at[src_v.at[j]], rows_v, sem).wait()
            pltpu.sync_copy(rows_v, acc_sh.at[dst_v.at[j]], add=True)

        plsc.subcore_barrier()

        pltpu.sync_copy(acc_sh.at[pl.ds(r0, RPS)],
                        acc_out.at[c, pl.ds(r0, RPS)])

    return k(data, src4, dst4, zacc)


_BN = 1000


def _tc1_body(x_ref, a1_ref, cnt_ref, w1s_ref, w1n_ref, b1_ref,
              w2c_ref, b2c_ref, u_ref):
    cnt = cnt_ref[:, 0:1] + cnt_ref[:, 1:2]
    mean1 = (a1_ref[0] + a1_ref[1]) / jnp.maximum(cnt, 1.0)
    s1 = jnp.dot(x_ref[...], w1s_ref[...], preferred_element_type=jnp.float32)
    m1 = jnp.dot(mean1, w1n_ref[...], preferred_element_type=jnp.float32)
    h = jax.nn.relu(s1 + m1 + b1_ref[...])
    u = jnp.dot(h, w2c_ref[...], preferred_element_type=jnp.float32)
    u_ref[...] = u + b2c_ref[...]


def _tc1(x, a1, cnt16, W1_self, W1_neigh, b1, W2cat, b2cat):
    grid = (N // _BN,)
    return pl.pallas_call(
        _tc1_body,
        grid=grid,
        in_specs=[
            pl.BlockSpec((_BN, F), lambda i: (i, 0)),
            pl.BlockSpec((NC, _BN, F), lambda i: (0, i, 0)),
            pl.BlockSpec((_BN, NC), lambda i: (i, 0)),
            pl.BlockSpec((F, H), lambda i: (0, 0)),
            pl.BlockSpec((F, H), lambda i: (0, 0)),
            pl.BlockSpec((1, H), lambda i: (0, 0)),
            pl.BlockSpec((H, 2 * W2P), lambda i: (0, 0)),
            pl.BlockSpec((1, 2 * W2P), lambda i: (0, 0)),
        ],
        out_specs=pl.BlockSpec((_BN, 2 * W2P), lambda i: (i, 0)),
        out_shape=jax.ShapeDtypeStruct((N, 2 * W2P), jnp.float32),
    )(x, a1, cnt16, W1_self, W1_neigh, b1, W2cat, b2cat)


def _tc2_body(u_ref, a2_ref, cnt_ref, out_ref):
    cnt = cnt_ref[:, 0:1] + cnt_ref[:, 1:2]
    mean2 = (a2_ref[0, :, :W2P] + a2_ref[1, :, :W2P]) / jnp.maximum(cnt, 1.0)
    t = u_ref[:, W2P:] + mean2
    col = lax.broadcasted_iota(jnp.int32, (_BN, W2P), 1)
    valid = col < C
    tm = jnp.where(valid, t, -jnp.inf)
    m = jnp.max(tm, axis=-1, keepdims=True)
    e = jnp.where(valid, jnp.exp(t - m), 0.0)
    lse = jnp.log(jnp.sum(e, axis=-1, keepdims=True))
    out_ref[...] = (t - m - lse)[:, :C]


def _tc2(u, a2, cnt16):
    grid = (N // _BN,)
    return pl.pallas_call(
        _tc2_body,
        grid=grid,
        in_specs=[
            pl.BlockSpec((_BN, 2 * W2P), lambda i: (i, 0)),
            pl.BlockSpec((NC, _BN, 2 * W2P), lambda i: (0, i, 0)),
            pl.BlockSpec((_BN, NC), lambda i: (i, 0)),
        ],
        out_specs=pl.BlockSpec((_BN, C), lambda i: (i, 0)),
        out_shape=jax.ShapeDtypeStruct((N, C), jnp.float32),
    )(u, a2, cnt16)


def kernel(x, adj, W1_self, W1_neigh, b1, W2_self, W2_neigh, b2):
    src, dst = adj[0], adj[1]
    pad = E_PAD - E
    pad_ar = jnp.arange(pad, dtype=jnp.int32)
    src_p = jnp.concatenate([src, pad_ar % N])
    dst_p = jnp.concatenate([dst, TRASH + pad_ar % (N_ACC - N)])
    src4 = src_p.reshape(NC, NS, K, G)
    dst4 = dst_p.reshape(NC, NS, K, G)

    zacc1 = jnp.zeros((N_ACC, F), jnp.float32)

    cnt2 = _seg_cnt(dst4).T[:N]
    a1 = _seg_sum(x, src4, dst4, zacc1, F)[:, :N]

    W2cat = jnp.concatenate([
        jnp.pad(W2_neigh, ((0, 0), (0, W2P - C))),
        jnp.pad(W2_self, ((0, 0), (0, W2P - C))),
    ], axis=1)
    b2cat = jnp.concatenate(
        [jnp.zeros((W2P,), jnp.float32), jnp.pad(b2, (0, W2P - C))]
    ).reshape(1, 2 * W2P)

    u = _tc1(x, a1, cnt2, W1_self, W1_neigh, b1.reshape(1, H),
             W2cat, b2cat)

    a2 = _seg_sum(u, src4, dst4, zacc1, 2 * W2P)[:, :N]

    return _tc2(u, a2, cnt2)
